# baseline (speedup 1.0000x reference)
import jax
import jax.numpy as jnp
from jax import lax
from jax.experimental import pallas as pl
from jax.experimental.pallas import tpu as pltpu
from jax.experimental.pallas import tpu_sc as plsc

_COL = 42
_ROWS = 16384
_NCOLS = 100
_INFO = plsc.get_sparse_core_info()
_NC = _INFO.num_cores
_NS = _INFO.num_subcores
_NW = _NC * _NS
_RPW = _ROWS // _NW
_NCHUNK = 4
_CR = _RPW // _NCHUNK


def _sc_select_column(x):
    mesh = plsc.VectorSubcoreMesh(core_axis_name="c", subcore_axis_name="s")

    @pl.kernel(
        out_type=jax.ShapeDtypeStruct((_ROWS,), jnp.float32),
        mesh=mesh,
        compiler_params=pltpu.CompilerParams(
            needs_layout_passes=False,
            disable_bounds_checks=True,
            disable_semaphore_checks=True,
            skip_device_barrier=True,
        ),
        scratch_types=[
            pltpu.VMEM((_RPW, _NCOLS), jnp.float32),
            pltpu.VMEM((_RPW,), jnp.float32),
            [pltpu.SemaphoreType.DMA] * _NCHUNK,
        ],
    )
    def k(x_hbm, out_hbm, slab_v, out_v, sems):
        wid = lax.axis_index("s") * _NC + lax.axis_index("c")
        base = wid * _RPW
        copies = [
            pltpu.async_copy(
                x_hbm.at[pl.ds(base + c * _CR, _CR)],
                slab_v.at[pl.ds(c * _CR, _CR)],
                sems[c],
            )
            for c in range(_NCHUNK)
        ]
        cols = jnp.full((16,), _COL, jnp.int32)
        iota = lax.iota(jnp.int32, 16)
        for c in range(_NCHUNK):
            copies[c].wait()
            for j in range(_CR // 16):
                rows = iota + (c * _CR + j * 16)
                v = plsc.load_gather(slab_v, [rows, cols])
                out_v[pl.ds(c * _CR + j * 16, 16)] = v
        pltpu.sync_copy(out_v, out_hbm.at[pl.ds(base, _RPW)])

    return k(x)


def kernel(x):
    if x.ndim == 1:
        x = x[None, :]
    return _sc_select_column(x.astype(jnp.float32)).reshape(_ROWS, 1)


# speedup vs baseline: 1.0036x; 1.0036x over previous
import jax
import jax.numpy as jnp
from jax import lax
from jax.experimental import pallas as pl
from jax.experimental.pallas import tpu as pltpu
from jax.experimental.pallas import tpu_sc as plsc

_COL = 42
_ROWS = 16384
_NCOLS = 100
_INFO = plsc.get_sparse_core_info()
_NC = _INFO.num_cores
_NS = _INFO.num_subcores
_NW = _NC * _NS
_RPW = _ROWS // _NW
_NCHUNK = 16
_CR = _RPW // _NCHUNK


def _sc_select_column(x):
    mesh = plsc.VectorSubcoreMesh(core_axis_name="c", subcore_axis_name="s")

    @pl.kernel(
        out_type=jax.ShapeDtypeStruct((_ROWS,), jnp.float32),
        mesh=mesh,
        compiler_params=pltpu.CompilerParams(
            needs_layout_passes=False,
            disable_bounds_checks=True,
            disable_semaphore_checks=True,
            skip_device_barrier=True,
        ),
        scratch_types=[
            pltpu.VMEM((_RPW, _NCOLS), jnp.float32),
            pltpu.VMEM((_RPW,), jnp.float32),
            [pltpu.SemaphoreType.DMA] * _NCHUNK,
        ],
    )
    def k(x_hbm, out_hbm, slab_v, out_v, sems):
        wid = lax.axis_index("s") * _NC + lax.axis_index("c")
        base = wid * _RPW
        copies = [
            pltpu.async_copy(
                x_hbm.at[pl.ds(base + c * _CR, _CR)],
                slab_v.at[pl.ds(c * _CR, _CR)],
                sems[c],
            )
            for c in range(_NCHUNK)
        ]
        cols = jnp.full((16,), _COL, jnp.int32)
        iota = lax.iota(jnp.int32, 16)
        for c in range(_NCHUNK):
            copies[c].wait()
            for j in range(_CR // 16):
                rows = iota + (c * _CR + j * 16)
                v = plsc.load_gather(slab_v, [rows, cols])
                out_v[pl.ds(c * _CR + j * 16, 16)] = v
        pltpu.sync_copy(out_v, out_hbm.at[pl.ds(base, _RPW)])

    return k(x)


def kernel(x):
    if x.ndim == 1:
        x = x[None, :]
    return _sc_select_column(x.astype(jnp.float32)).reshape(_ROWS, 1)


# speedup vs baseline: 1.0239x; 1.0202x over previous
import jax
import jax.numpy as jnp
from jax import lax
from jax.experimental import pallas as pl
from jax.experimental.pallas import tpu as pltpu
from jax.experimental.pallas import tpu_sc as plsc

_COL = 42
_ROWS = 16384
_NCOLS = 100
_INFO = plsc.get_sparse_core_info()
_NC = _INFO.num_cores
_NS = _INFO.num_subcores
_NW = _NC * _NS
_RPW = _ROWS // _NW


def _sc_select_column(x):
    mesh = plsc.VectorSubcoreMesh(core_axis_name="c", subcore_axis_name="s")

    @pl.kernel(
        out_type=jax.ShapeDtypeStruct((_ROWS,), jnp.float32),
        mesh=mesh,
        compiler_params=pltpu.CompilerParams(
            needs_layout_passes=False,
        ),
        scratch_types=[
            pltpu.VMEM((_RPW, _NCOLS), jnp.float32),
            pltpu.VMEM((_RPW,), jnp.float32),
        ],
    )
    def k(x_hbm, out_hbm, slab_v, out_v):
        wid = lax.axis_index("s") * _NC + lax.axis_index("c")
        base = wid * _RPW
        pltpu.sync_copy(x_hbm.at[pl.ds(base, _RPW)], slab_v)
        cols = jnp.full((16,), _COL, jnp.int32)
        for j in range(_RPW // 16):
            rows = lax.iota(jnp.int32, 16) + j * 16
            v = plsc.load_gather(slab_v, [rows, cols])
            out_v[pl.ds(j * 16, 16)] = v
        pltpu.sync_copy(out_v, out_hbm.at[pl.ds(base, _RPW)])

    return k(x)


def kernel(x):
    if x.ndim == 1:
        x = x[None, :]
    return _sc_select_column(x.astype(jnp.float32)).reshape(_ROWS, 1)
